# Initial kernel scaffold; baseline (speedup 1.0000x reference)
import jax
import jax.numpy as jnp
from jax.experimental import pallas as pl


def kernel(inputs, emb_weights):
    raise NotImplementedError("write your pallas kernel here")


# speedup vs baseline: 3.8718x; 3.8718x over previous
import functools

import jax
import jax.numpy as jnp
from jax.experimental import pallas as pl
from jax.experimental.pallas import tpu as pltpu

_NUM_EMBEDDINGS = 1024
_EMBED_DIM = 768
_NUM_HEADS = 4
_DH = _EMBED_DIM // _NUM_HEADS
_COMMITMENT_COST = 0.25

_BLOCK = 512


def _vq_kernel(x_ref, w_ref, q_ref, codes_ref, loss_ref):
    x = x_ref[...]
    acc = jnp.zeros((), dtype=jnp.float32)
    code_iota = jax.lax.broadcasted_iota(jnp.int32, (1, _NUM_EMBEDDINGS), 1)
    for h in range(_NUM_HEADS):
        xh = x[:, h * _DH:(h + 1) * _DH]
        wh = w_ref[h]
        m = jax.lax.dot_general(
            xh, wh, (((1,), (1,)), ((), ())),
            preferred_element_type=jnp.float32)
        a = jnp.sum(xh * xh, axis=1, keepdims=True)
        b = jnp.sum(wh * wh, axis=1)
        d = (a + b) - 2.0 * m
        dmin = jnp.min(d, axis=1, keepdims=True)
        idx = jnp.min(
            jnp.where(d == dmin, code_iota, _NUM_EMBEDDINGS),
            axis=1).astype(jnp.int32)
        codes_ref[h, :] = idx
        onehot = (code_iota == idx[:, None]).astype(jnp.float32)
        qh = jax.lax.dot_general(
            onehot, wh, (((1,), (0,)), ((), ())),
            preferred_element_type=jnp.float32)
        q_ref[:, h * _DH:(h + 1) * _DH] = qh
        r = qh - xh
        acc = acc + jnp.sum(r * r)
    loss_ref[...] = acc.reshape(1, 1, 1)


@jax.jit
def kernel(inputs, emb_weights):
    input_shape = inputs.shape
    n = input_shape[0] * input_shape[1]
    x = inputs.reshape(n, _EMBED_DIM)
    nblocks = n // _BLOCK

    q, codes, loss_parts = pl.pallas_call(
        _vq_kernel,
        grid=(nblocks,),
        in_specs=[
            pl.BlockSpec((_BLOCK, _EMBED_DIM), lambda i: (i, 0)),
            pl.BlockSpec((_NUM_HEADS, _NUM_EMBEDDINGS, _DH),
                         lambda i: (0, 0, 0)),
        ],
        out_specs=[
            pl.BlockSpec((_BLOCK, _EMBED_DIM), lambda i: (i, 0)),
            pl.BlockSpec((_NUM_HEADS, _BLOCK), lambda i: (0, i)),
            pl.BlockSpec((1, 1, 1), lambda i: (i, 0, 0)),
        ],
        out_shape=[
            jax.ShapeDtypeStruct((n, _EMBED_DIM), jnp.float32),
            jax.ShapeDtypeStruct((_NUM_HEADS, n), jnp.int32),
            jax.ShapeDtypeStruct((nblocks, 1, 1), jnp.float32),
        ],
        compiler_params=pltpu.CompilerParams(
            dimension_semantics=("arbitrary",)),
    )(x, emb_weights)

    numel = n * _EMBED_DIM
    loss = jnp.sum(loss_parts) * (_COMMITMENT_COST / numel)
    quantized = q.reshape(input_shape)
    vq_codes = codes.reshape(_NUM_HEADS, n, 1)
    return loss, quantized, vq_codes


# speedup vs baseline: 4.0761x; 1.0528x over previous
import functools

import jax
import jax.numpy as jnp
from jax.experimental import pallas as pl
from jax.experimental.pallas import tpu as pltpu

_NUM_EMBEDDINGS = 1024
_EMBED_DIM = 768
_NUM_HEADS = 4
_DH = _EMBED_DIM // _NUM_HEADS
_COMMITMENT_COST = 0.25

_BLOCK = 512


def _vq_kernel(x_ref, w_ref, q_ref, codes_ref, loss_ref, b_scr):
    @pl.when(pl.program_id(0) == 0)
    def _():
        for h in range(_NUM_HEADS):
            wh = w_ref[h]
            b_scr[h] = jnp.sum(wh * wh, axis=1)[None, :]

    x = x_ref[...]
    acc = jnp.zeros((), dtype=jnp.float32)
    code_iota = jax.lax.broadcasted_iota(jnp.int32, (1, _NUM_EMBEDDINGS), 1)
    for h in range(_NUM_HEADS):
        xh = x[:, h * _DH:(h + 1) * _DH]
        wh = w_ref[h]
        m = jax.lax.dot_general(
            xh, wh, (((1,), (1,)), ((), ())),
            preferred_element_type=jnp.float32)
        a = jnp.sum(xh * xh, axis=1, keepdims=True)
        d = (a + b_scr[h]) - 2.0 * m
        dmin = jnp.min(d, axis=1, keepdims=True)
        idx = jnp.min(
            jnp.where(d == dmin, code_iota, _NUM_EMBEDDINGS),
            axis=1).astype(jnp.int32)
        codes_ref[h, :] = idx
        onehot = (code_iota == idx[:, None]).astype(jnp.float32)
        qh = jax.lax.dot_general(
            onehot, wh, (((1,), (0,)), ((), ())),
            preferred_element_type=jnp.float32)
        q_ref[:, h * _DH:(h + 1) * _DH] = qh
        acc = acc + jnp.sum(dmin)
    loss_ref[...] = acc.reshape(1, 1, 1)


@jax.jit
def kernel(inputs, emb_weights):
    input_shape = inputs.shape
    n = input_shape[0] * input_shape[1]
    x = inputs.reshape(n, _EMBED_DIM)
    nblocks = n // _BLOCK

    q, codes, loss_parts = pl.pallas_call(
        _vq_kernel,
        grid=(nblocks,),
        in_specs=[
            pl.BlockSpec((_BLOCK, _EMBED_DIM), lambda i: (i, 0)),
            pl.BlockSpec((_NUM_HEADS, _NUM_EMBEDDINGS, _DH),
                         lambda i: (0, 0, 0)),
        ],
        out_specs=[
            pl.BlockSpec((_BLOCK, _EMBED_DIM), lambda i: (i, 0)),
            pl.BlockSpec((_NUM_HEADS, _BLOCK), lambda i: (0, i)),
            pl.BlockSpec((1, 1, 1), lambda i: (i, 0, 0)),
        ],
        out_shape=[
            jax.ShapeDtypeStruct((n, _EMBED_DIM), jnp.float32),
            jax.ShapeDtypeStruct((_NUM_HEADS, n), jnp.int32),
            jax.ShapeDtypeStruct((nblocks, 1, 1), jnp.float32),
        ],
        scratch_shapes=[pltpu.VMEM((_NUM_HEADS, 1, _NUM_EMBEDDINGS),
                                   jnp.float32)],
        compiler_params=pltpu.CompilerParams(
            dimension_semantics=("arbitrary",)),
    )(x, emb_weights)

    numel = n * _EMBED_DIM
    loss = jnp.sum(loss_parts) * (_COMMITMENT_COST / numel)
    quantized = q.reshape(input_shape)
    vq_codes = codes.reshape(_NUM_HEADS, n, 1)
    return loss, quantized, vq_codes
